# baseline (device time: 12675 ns/iter reference)
import jax
import jax.numpy as jnp
from jax import lax
from jax.experimental import pallas as pl
from jax.experimental.pallas import tpu as pltpu

N_BLOCKS = 8
LANES = 128


def kernel(x):
    m, n = x.shape
    m_blk = m // N_BLOCKS
    g_blk = m_blk // LANES
    g_tot = m // LANES
    g_half = g_tot // 2

    def body(x_ref, out_ref, acc_ref, recv_ref, send_sems, recv_sems):
        i = pl.program_id(0)
        my_x = lax.axis_index("x")
        my_y = lax.axis_index("y")
        peer = (my_x, 1 - my_y)

        def half_rdma(h):
            return pltpu.make_async_remote_copy(
                src_ref=acc_ref.at[pl.ds(h * g_half, g_half)],
                dst_ref=recv_ref.at[pl.ds(h * g_half, g_half)],
                send_sem=send_sems.at[h],
                recv_sem=recv_sems.at[h],
                device_id=peer,
                device_id_type=pl.DeviceIdType.MESH,
            )

        def combine_unpack(h):
            rows = pl.ds(h * g_half, g_half)
            cmb = jnp.maximum(acc_ref[rows, :], recv_ref[rows, :])
            t = cmb.T
            for q in range(g_half):
                out_ref[(h * g_half + q) * LANES:
                        (h * g_half + q + 1) * LANES, :] = t[:, q:q + 1]

        @pl.when(i == 0)
        def _():
            barrier_sem = pltpu.get_barrier_semaphore()
            pl.semaphore_signal(
                barrier_sem, inc=1,
                device_id=peer, device_id_type=pl.DeviceIdType.MESH,
            )
            pl.semaphore_wait(barrier_sem, 1)

        fold = x_ref[:, 0:LANES]
        for c in range(1, n // LANES):
            fold = jnp.maximum(fold, x_ref[:, c * LANES:(c + 1) * LANES])

        for k in range(g_blk):
            t = fold[k * LANES:(k + 1) * LANES, :].T
            acc_ref[pl.ds(i * g_blk + k, 1), :] = jnp.max(
                t, axis=0, keepdims=True
            )

        @pl.when(i == N_BLOCKS // 2 - 1)
        def _():
            half_rdma(0).start()

        @pl.when(i == N_BLOCKS - 1)
        def _():
            half_rdma(1).start()
            half_rdma(0).wait()
            combine_unpack(0)
            half_rdma(1).wait()
            combine_unpack(1)

    return pl.pallas_call(
        body,
        grid=(N_BLOCKS,),
        out_shape=jax.ShapeDtypeStruct((m, 1), x.dtype),
        in_specs=[
            pl.BlockSpec((m_blk, n), lambda i: (i, 0), memory_space=pltpu.VMEM)
        ],
        out_specs=pl.BlockSpec((m, 1), lambda i: (0, 0), memory_space=pltpu.VMEM),
        scratch_shapes=[
            pltpu.VMEM((g_tot, LANES), x.dtype),
            pltpu.VMEM((g_tot, LANES), x.dtype),
            pltpu.SemaphoreType.DMA((2,)),
            pltpu.SemaphoreType.DMA((2,)),
        ],
        compiler_params=pltpu.CompilerParams(
            collective_id=0,
            dimension_semantics=("arbitrary",),
        ),
    )(x)


# device time: 12095 ns/iter; 1.0480x vs baseline; 1.0480x over previous
import jax
import jax.numpy as jnp
from jax import lax
from jax.experimental import pallas as pl
from jax.experimental.pallas import tpu as pltpu

N_BLOCKS = 4
LANES = 128


def kernel(x):
    m, n = x.shape
    m_blk = m // N_BLOCKS
    g_blk = m_blk // LANES
    g_tot = m // LANES

    def body(x_ref, out_ref, acc_ref, recv_ref, send_sems, recv_sems):
        i = pl.program_id(0)
        my_x = lax.axis_index("x")
        my_y = lax.axis_index("y")
        peer = (my_x, 1 - my_y)

        def block_rdma(b):
            return pltpu.make_async_remote_copy(
                src_ref=acc_ref.at[pl.ds(b * g_blk, g_blk)],
                dst_ref=recv_ref.at[pl.ds(b * g_blk, g_blk)],
                send_sem=send_sems.at[b],
                recv_sem=recv_sems.at[b],
                device_id=peer,
                device_id_type=pl.DeviceIdType.MESH,
            )

        def combine_unpack(b):
            rows = pl.ds(b * g_blk, g_blk)
            cmb = jnp.maximum(acc_ref[rows, :], recv_ref[rows, :])
            t = cmb.T
            for q in range(g_blk):
                out_ref[pl.ds((b * g_blk + q) * LANES, LANES), :] = t[:, q:q + 1]

        @pl.when(i == 0)
        def _():
            barrier_sem = pltpu.get_barrier_semaphore()
            pl.semaphore_signal(
                barrier_sem, inc=1,
                device_id=peer, device_id_type=pl.DeviceIdType.MESH,
            )
            pl.semaphore_wait(barrier_sem, 1)

        fold = x_ref[:, 0:LANES]
        for c in range(1, n // LANES):
            fold = jnp.maximum(fold, x_ref[:, c * LANES:(c + 1) * LANES])

        for k in range(g_blk):
            t = fold[k * LANES:(k + 1) * LANES, :].T
            acc_ref[pl.ds(i * g_blk + k, 1), :] = jnp.max(
                t, axis=0, keepdims=True
            )

        block_rdma(i).start()

        @pl.when(i == N_BLOCKS - 1)
        def _():
            for b in range(N_BLOCKS - 1):
                block_rdma(b).wait()
                combine_unpack(b)
            block_rdma(N_BLOCKS - 1).wait()
            combine_unpack(N_BLOCKS - 1)

    return pl.pallas_call(
        body,
        grid=(N_BLOCKS,),
        out_shape=jax.ShapeDtypeStruct((m, 1), x.dtype),
        in_specs=[
            pl.BlockSpec((m_blk, n), lambda i: (i, 0), memory_space=pltpu.VMEM)
        ],
        out_specs=pl.BlockSpec((m, 1), lambda i: (0, 0), memory_space=pltpu.VMEM),
        scratch_shapes=[
            pltpu.VMEM((g_tot, LANES), x.dtype),
            pltpu.VMEM((g_tot, LANES), x.dtype),
            pltpu.SemaphoreType.DMA((N_BLOCKS,)),
            pltpu.SemaphoreType.DMA((N_BLOCKS,)),
        ],
        compiler_params=pltpu.CompilerParams(
            collective_id=0,
            dimension_semantics=("arbitrary",),
        ),
    )(x)
